# baseline (speedup 1.0000x reference)
import jax
import jax.numpy as jnp
from jax.experimental import pallas as pl
from jax.experimental.pallas import tpu as pltpu

_NBUF = 8
_W = 4


def _schedule(n):
    small, nsmall = 240, 4
    if n <= 2 * small * nsmall:
        c = max(8, n // 16 // 8 * 8)
        sizes = [c] * (n // c)
        if n % c:
            sizes.append(n % c)
        return sizes
    mid = n - 2 * small * nsmall
    nbig = max(1, mid // 4400)
    big = mid // nbig // 8 * 8
    sizes = [small] * nsmall + [big] * nbig + [small] * nsmall
    rem = n - sum(sizes)
    assert rem >= 0 and rem % 8 == 0
    if rem:
        sizes.insert(nsmall, rem)
    return sizes


def _make_relay(sizes, d):
    offs = [0]
    for s in sizes:
        offs.append(offs[-1] + s)
    nchunk = len(sizes)
    bufrows = max(sizes)

    def _relay(x_hbm, o_hbm, bufs, in_sems, out_sems):
        def in_cp(i):
            b = i % _NBUF
            return pltpu.make_async_copy(
                x_hbm.at[pl.ds(offs[i], sizes[i]), :],
                bufs.at[b, pl.ds(0, sizes[i]), :],
                in_sems.at[b])

        def out_cp(i):
            b = i % _NBUF
            return pltpu.make_async_copy(
                bufs.at[b, pl.ds(0, sizes[i]), :],
                o_hbm.at[pl.ds(offs[i], sizes[i]), :],
                out_sems.at[b])

        for i in range(min(_W, nchunk)):
            in_cp(i).start()
        waited_out = 0
        for i in range(nchunk):
            nxt = i + _W
            if nxt < nchunk:
                prev = nxt - _NBUF
                if prev >= 0:
                    out_cp(prev).wait()
                    waited_out = prev + 1
                in_cp(nxt).start()
            in_cp(i).wait()
            out_cp(i).start()
        for i in range(waited_out, nchunk):
            out_cp(i).wait()

    return _relay, nchunk, bufrows


def kernel(x, u):
    n, d = x.shape
    sizes = _schedule(n)
    relay, nchunk, bufrows = _make_relay(sizes, d)
    return pl.pallas_call(
        relay,
        in_specs=[pl.BlockSpec(memory_space=pl.ANY)],
        out_specs=pl.BlockSpec(memory_space=pl.ANY),
        out_shape=jax.ShapeDtypeStruct((n, d), x.dtype),
        scratch_shapes=[
            pltpu.VMEM((_NBUF, bufrows, d), jnp.float32),
            pltpu.SemaphoreType.DMA((_NBUF,)),
            pltpu.SemaphoreType.DMA((_NBUF,)),
        ],
    )(x)


# speedup vs baseline: 1.0087x; 1.0087x over previous
import jax
import jax.numpy as jnp
from jax.experimental import pallas as pl
from jax.experimental.pallas import tpu as pltpu

_NBUF = 8
_W = 4


def _schedule(n):
    small, nsmall = 400, 4
    if n <= 2 * small * nsmall:
        c = max(8, n // 16 // 8 * 8)
        sizes = [c] * (n // c)
        if n % c:
            sizes.append(n % c)
        return sizes
    mid = n - 2 * small * nsmall
    nbig = max(1, mid // 4400)
    big = mid // nbig // 8 * 8
    sizes = [small] * nsmall + [big] * nbig + [small] * nsmall
    rem = n - sum(sizes)
    assert rem >= 0 and rem % 8 == 0
    if rem:
        sizes.insert(nsmall, rem)
    return sizes


def _make_relay(sizes, d):
    offs = [0]
    for s in sizes:
        offs.append(offs[-1] + s)
    nchunk = len(sizes)
    bufrows = max(sizes)

    def _relay(x_hbm, o_hbm, bufs, in_sems, out_sems):
        def in_cp(i):
            b = i % _NBUF
            return pltpu.make_async_copy(
                x_hbm.at[pl.ds(offs[i], sizes[i]), :],
                bufs.at[b, pl.ds(0, sizes[i]), :],
                in_sems.at[b])

        def out_cp(i):
            b = i % _NBUF
            return pltpu.make_async_copy(
                bufs.at[b, pl.ds(0, sizes[i]), :],
                o_hbm.at[pl.ds(offs[i], sizes[i]), :],
                out_sems.at[b])

        for i in range(min(_W, nchunk)):
            in_cp(i).start()
        waited_out = 0
        for i in range(nchunk):
            nxt = i + _W
            if nxt < nchunk:
                prev = nxt - _NBUF
                if prev >= 0:
                    out_cp(prev).wait()
                    waited_out = prev + 1
                in_cp(nxt).start()
            in_cp(i).wait()
            out_cp(i).start()
        for i in range(waited_out, nchunk):
            out_cp(i).wait()

    return _relay, nchunk, bufrows


def kernel(x, u):
    n, d = x.shape
    sizes = _schedule(n)
    relay, nchunk, bufrows = _make_relay(sizes, d)
    return pl.pallas_call(
        relay,
        in_specs=[pl.BlockSpec(memory_space=pl.ANY)],
        out_specs=pl.BlockSpec(memory_space=pl.ANY),
        out_shape=jax.ShapeDtypeStruct((n, d), x.dtype),
        scratch_shapes=[
            pltpu.VMEM((_NBUF, bufrows, d), jnp.float32),
            pltpu.SemaphoreType.DMA((_NBUF,)),
            pltpu.SemaphoreType.DMA((_NBUF,)),
        ],
    )(x)


# speedup vs baseline: 1.0091x; 1.0004x over previous
import jax
import jax.numpy as jnp
from jax.experimental import pallas as pl
from jax.experimental.pallas import tpu as pltpu

_NBUF = 8
_W = 4


def _schedule(n):
    small, nsmall = 400, 4
    if n <= 2 * small * nsmall:
        c = max(8, n // 16 // 8 * 8)
        sizes = [c] * (n // c)
        if n % c:
            sizes.append(n % c)
        return sizes
    mid = n - 2 * small * nsmall
    nbig = max(1, mid // 4400)
    big = mid // nbig // 8 * 8
    sizes = [small] * nsmall + [big] * nbig + [small] * nsmall
    rem = n - sum(sizes)
    assert rem >= 0 and rem % 8 == 0
    if rem:
        sizes.insert(nsmall, rem)
    return sizes


def _make_relay(sizes):
    offs = [0]
    for s in sizes:
        offs.append(offs[-1] + s)
    nchunk = len(sizes)

    def _relay(x_hbm, o_hbm, bufs, in_sems, out_sems):
        def in_cp(i):
            b = i % _NBUF
            return pltpu.make_async_copy(
                x_hbm.at[pl.ds(offs[i], sizes[i]), :],
                bufs.at[b, pl.ds(0, sizes[i]), :],
                in_sems.at[b])

        def out_cp(i):
            b = i % _NBUF
            return pltpu.make_async_copy(
                bufs.at[b, pl.ds(0, sizes[i]), :],
                o_hbm.at[pl.ds(offs[i], sizes[i]), :],
                out_sems.at[b])

        for i in range(min(_W, nchunk)):
            in_cp(i).start()
        waited_out = 0
        for i in range(nchunk):
            nxt = i + _W
            if nxt < nchunk:
                prev = nxt - _NBUF
                if prev >= 0:
                    out_cp(prev).wait()
                    waited_out = prev + 1
                in_cp(nxt).start()
            in_cp(i).wait()
            out_cp(i).start()
        for i in range(waited_out, nchunk):
            out_cp(i).wait()

    return _relay


def kernel(x, u):
    n, d = x.shape
    sizes = _schedule(n)
    relay = _make_relay(sizes)
    bufrows = max(sizes)
    return pl.pallas_call(
        relay,
        in_specs=[pl.BlockSpec(memory_space=pl.ANY)],
        out_specs=pl.BlockSpec(memory_space=pl.ANY),
        out_shape=jax.ShapeDtypeStruct((n, d), x.dtype),
        scratch_shapes=[
            pltpu.VMEM((_NBUF, bufrows, d), jnp.float32),
            pltpu.SemaphoreType.DMA((_NBUF,)),
            pltpu.SemaphoreType.DMA((_NBUF,)),
        ],
    )(x)
